# baseline (speedup 1.0000x reference)
import functools

import jax
import jax.numpy as jnp
from jax import lax
from jax.experimental import pallas as pl
from jax.experimental.pallas import tpu as pltpu
from jax.experimental.pallas import tpu_sc as plsc

_LAMBDA_C = 1.0
_IDX_CHUNK = 128


@functools.lru_cache(maxsize=None)
def _build_sc_partials(batch: int, dim: int):
    info = plsc.get_sparse_core_info()
    nc, ns, lanes = info.num_cores, info.num_subcores, info.num_lanes
    nw = nc * ns
    b_per_w = batch // nw
    assert batch % nw == 0 and dim % lanes == 0
    assert b_per_w % _IDX_CHUNK == 0
    n_chunks = b_per_w // _IDX_CHUNK

    mesh = plsc.VectorSubcoreMesh(core_axis_name="c", subcore_axis_name="s")

    @functools.partial(
        pl.kernel,
        mesh=mesh,
        out_type=jax.ShapeDtypeStruct((nw, lanes), jnp.float32),
        compiler_params=pltpu.CompilerParams(use_tc_tiling_on_sc=False),
        scratch_types=[
            pltpu.VMEM((n_chunks, _IDX_CHUNK), jnp.int32),
            pltpu.VMEM((b_per_w, dim), jnp.float32),
            pltpu.VMEM((b_per_w, dim), jnp.float32),
            pltpu.VMEM((lanes,), jnp.float32),
            pltpu.SemaphoreType.DMA,
        ],
    )
    def sc_partials(y_hbm, hidden_hbm, centers_hbm, out_hbm,
                    idx_v, ctr_v, hid_v, acc_v, sem):
        wid = lax.axis_index("s") * nc + lax.axis_index("c")
        base = wid * b_per_w
        pltpu.sync_copy(y_hbm.at[wid], idx_v)
        copies = []
        for j in range(n_chunks):
            copies.append(pltpu.async_copy(
                centers_hbm.at[idx_v.at[j]],
                ctr_v.at[pl.ds(j * _IDX_CHUNK, _IDX_CHUNK)],
                sem))
        pltpu.sync_copy(hidden_hbm.at[pl.ds(base, b_per_w)], hid_v)
        for cp in copies:
            cp.wait()

        def body(r, acc):
            for j in range(dim // lanes):
                h = hid_v[r, pl.ds(j * lanes, lanes)]
                c = ctr_v[r, pl.ds(j * lanes, lanes)]
                d = h - c
                acc = acc + d * d
            return acc

        acc = lax.fori_loop(0, b_per_w, body, jnp.zeros((lanes,), jnp.float32))
        acc_v[...] = acc
        pltpu.sync_copy(acc_v, out_hbm.at[wid])

    return sc_partials, nw, lanes, n_chunks


def _finish_body(scale_sq, p_ref, o_ref):
    s = jnp.sum(p_ref[...])
    o_ref[...] = jnp.broadcast_to(jnp.sqrt(s * scale_sq), (1, 1))


def kernel(y, hidden, centers):
    batch, dim = hidden.shape
    sc_partials, nw, lanes, n_chunks = _build_sc_partials(batch, dim)
    y_grp = y.astype(jnp.int32).reshape(nw, n_chunks, _IDX_CHUNK)
    partials = sc_partials(y_grp, hidden, centers)
    scale = _LAMBDA_C / 2.0 / batch
    loss = pl.pallas_call(
        functools.partial(_finish_body, scale * scale),
        out_shape=jax.ShapeDtypeStruct((1, 1), jnp.float32),
    )(partials)
    return loss[0, 0]


# speedup vs baseline: 1.6995x; 1.6995x over previous
import functools

import jax
import jax.numpy as jnp
from jax import lax
from jax.experimental import pallas as pl
from jax.experimental.pallas import tpu as pltpu
from jax.experimental.pallas import tpu_sc as plsc

_LAMBDA_C = 1.0


@functools.lru_cache(maxsize=None)
def _build_sc_partials(batch: int, dim: int):
    info = plsc.get_sparse_core_info()
    nc, ns, lanes = info.num_cores, info.num_subcores, info.num_lanes
    nw = nc * ns
    b_per_w = batch // nw
    assert batch % nw == 0 and dim % lanes == 0

    mesh = plsc.VectorSubcoreMesh(core_axis_name="c", subcore_axis_name="s")

    @functools.partial(
        pl.kernel,
        mesh=mesh,
        out_type=jax.ShapeDtypeStruct((nw * lanes,), jnp.float32),
        scratch_types=[
            pltpu.VMEM((b_per_w,), jnp.int32),
            pltpu.VMEM((b_per_w, 2 * dim), jnp.float32),
            pltpu.VMEM((lanes,), jnp.float32),
            pltpu.HBM((b_per_w, 2 * dim), jnp.float32),
            pltpu.SemaphoreType.DMA,
        ],
    )
    def sc_partials(y_hbm, hidden_hbm, centers_hbm, out_hbm,
                    idx_v, cat_v, acc_v, dummy_hbm, sem):
        wid = lax.axis_index("s") * nc + lax.axis_index("c")
        base = wid * b_per_w
        pltpu.sync_copy(y_hbm.at[pl.ds(base, b_per_w)], idx_v)

        def fire(g, carry):
            vec = idx_v[pl.ds(g * lanes, lanes)]
            for k in range(lanes):
                r = g * lanes + k
                pltpu.async_copy(centers_hbm.at[vec[k]],
                                 cat_v.at[r, pl.ds(0, dim)], sem)
                pltpu.async_copy(hidden_hbm.at[base + r],
                                 cat_v.at[r, pl.ds(dim, dim)], sem)
            return carry

        lax.fori_loop(0, b_per_w // lanes, fire, 0)
        pltpu.make_async_copy(dummy_hbm, cat_v, sem).wait()

        def body(r, acc):
            for j in range(dim // lanes):
                c = cat_v[r, pl.ds(j * lanes, lanes)]
                h = cat_v[r, pl.ds(dim + j * lanes, lanes)]
                d = h - c
                acc = acc + d * d
            return acc

        acc = lax.fori_loop(0, b_per_w, body, jnp.zeros((lanes,), jnp.float32))
        acc_v[...] = acc
        pltpu.sync_copy(acc_v, out_hbm.at[pl.ds(wid * lanes, lanes)])

    return sc_partials, nw, lanes


def _finish_body(scale_sq, p_ref, o_ref):
    s = jnp.sum(p_ref[...])
    o_ref[...] = jnp.broadcast_to(jnp.sqrt(s * scale_sq), (1, 1))


def kernel(y, hidden, centers):
    batch, dim = hidden.shape
    sc_partials, nw, lanes = _build_sc_partials(batch, dim)
    partials = sc_partials(y.astype(jnp.int32), hidden, centers)
    scale = _LAMBDA_C / 2.0 / batch
    loss = pl.pallas_call(
        functools.partial(_finish_body, scale * scale),
        out_shape=jax.ShapeDtypeStruct((1, 1), jnp.float32),
    )(partials)
    return loss[0, 0]
